# baseline (speedup 1.0000x reference)
import functools

import jax
import jax.numpy as jnp
from jax import lax
from jax.experimental import pallas as pl
from jax.experimental.pallas import tpu as pltpu

_B = 8
_HP = 16
_PP = _HP * _HP
_C = 256


def _mask_head_kernel(x_ref, wc_ref, wtc_ref, w5b_ref, bias_ref, out_ref):
    bm = x_ref.shape[0]
    ri = lax.broadcasted_iota(jnp.int32, (bm, 1), 0)
    local = ri & (_PP - 1)
    h = local >> 4
    w = local & (_HP - 1)
    interior = (h >= 1) & (h <= 14) & (w >= 1) & (w <= 14)

    dot = functools.partial(jnp.dot, preferred_element_type=jnp.float32)

    x = x_ref[...]
    for l in range(4):
        shifted = {-1: pltpu.roll(x, 1, 0), 0: x, 1: pltpu.roll(x, bm - 1, 0)}
        acc = None
        for r in (-1, 0, 1):
            p = None
            for c in (-1, 0, 1):
                t = (r + 1) * 3 + (c + 1)
                term = dot(shifted[c], wc_ref[l, t])
                p = term if p is None else p + term
            if r != 0:
                p = pltpu.roll(p, (-16 * r) % bm, 0)
            acc = p if acc is None else acc + p
        y = acc + bias_ref[l : l + 1, 0:_C]
        x = jnp.where(interior, jnp.maximum(y, 0.0), 0.0)

    z = jnp.maximum(dot(x, wtc_ref[...]) + bias_ref[4:5, :], 0.0)
    out = jax.nn.sigmoid(dot(z, w5b_ref[...]) + bias_ref[5:6, 0:12])
    out_ref[...] = out


def kernel(features, w1, b1, w2, b2, w3, b3, w4, b4, wt, bt, w5, b5):
    n = features.shape[0]
    b = _B
    assert n % b == 0
    bm = b * _PP

    xt = jnp.transpose(features, (0, 2, 3, 1))
    xpad = jnp.pad(xt, ((0, 0), (1, 1), (1, 1), (0, 0)))
    xrows = xpad.reshape(n * _PP, _C)

    wc = jnp.stack(
        [jnp.transpose(wl, (2, 3, 1, 0)).reshape(9, _C, _C) for wl in (w1, w2, w3, w4)]
    )
    wtc = jnp.transpose(wt, (0, 2, 3, 1)).reshape(_C, 4 * _C)
    w5m = jnp.transpose(w5[:, :, 0, 0])
    w5b = jnp.kron(jnp.eye(4, dtype=w5m.dtype), w5m)

    bias = jnp.zeros((8, 4 * _C), dtype=jnp.float32)
    bias = bias.at[0:4, 0:_C].set(jnp.stack([b1, b2, b3, b4]))
    bias = bias.at[4, :].set(jnp.tile(bt, 4))
    bias = bias.at[5, 0:12].set(jnp.tile(b5, 4))

    out = pl.pallas_call(
        _mask_head_kernel,
        grid=(n // b,),
        in_specs=[
            pl.BlockSpec((bm, _C), lambda i: (i, 0)),
            pl.BlockSpec((4, 9, _C, _C), lambda i: (0, 0, 0, 0)),
            pl.BlockSpec((_C, 4 * _C), lambda i: (0, 0)),
            pl.BlockSpec((4 * _C, 12), lambda i: (0, 0)),
            pl.BlockSpec((8, 4 * _C), lambda i: (0, 0)),
        ],
        out_specs=pl.BlockSpec((bm, 12), lambda i: (i, 0)),
        out_shape=jax.ShapeDtypeStruct((n * _PP, 12), jnp.float32),
        compiler_params=pltpu.CompilerParams(
            dimension_semantics=("parallel",),
        ),
    )(xrows, wc, wtc, w5b, bias)

    m = out.reshape(n, _HP, _HP, 2, 2, 3)[:, 1:15, 1:15]
    return m.transpose(0, 5, 1, 3, 2, 4).reshape(n, 3, 28, 28)
